# Initial kernel scaffold; baseline (speedup 1.0000x reference)
import jax
import jax.numpy as jnp
from jax.experimental import pallas as pl


def kernel(x, edge_index, batch, sheet_node_idx, sheet_features, params):
    raise NotImplementedError("write your pallas kernel here")


# speedup vs baseline: 1.0010x; 1.0010x over previous
import functools

import jax
import jax.numpy as jnp
import numpy as np
from jax.experimental import pallas as pl
from jax.experimental.pallas import tpu as pltpu

HEADS = 4
CH = 16
D = 64
F = 10


def _dot(a, b, hi=False):
    return jax.lax.dot_general(
        a, b, (((1,), (0,)), ((), ())),
        precision=jax.lax.Precision.HIGHEST if hi else None,
        preferred_element_type=jnp.float32)


def _ln(x, g, b):
    mu = x.mean(-1, keepdims=True)
    var = x.var(-1, keepdims=True)
    return (x - mu) / jnp.sqrt(var + 1e-5) * g + b


def _heads_body(se_ref, ge_ref, sf_ref,
                gW1, gb1, gg1, gbe1, gW2, gb2, gg2, gbe2,
                mWq, mbq, mWk, mbk, mWv, mbv, mWo, mbo,
                qW1, qb1, qg1, qbe1, qW2, qb2, qg2, qbe2, qW3, qb3,
                dW1, db1, dg1, dbe1, dW2, db2,
                qv_ref, dl_ref):
    se = se_ref[...]
    ge = ge_ref[...]
    sf = sf_ref[...]
    geo = jax.nn.relu(_ln(_dot(sf, gW1[...]) + gb1[...], gg1[...], gbe1[...]))
    geo = jax.nn.relu(_ln(_dot(geo, gW2[...]) + gb2[...], gg2[...], gbe2[...]))
    qh = _dot(se, mWq[...]) + mbq[...]
    kh = _dot(geo, mWk[...]) + mbk[...]
    vh = _dot(geo, mWv[...]) + mbv[...]
    outs = []
    scale = 1.0 / np.sqrt(CH)
    for h in range(HEADS):
        q1 = qh[:, h * CH:(h + 1) * CH]
        k1 = kh[:, h * CH:(h + 1) * CH]
        v1 = vh[:, h * CH:(h + 1) * CH]
        att = _dot(q1, k1.T) * scale
        att = jax.nn.softmax(att, axis=-1)
        outs.append(_dot(att, v1))
    o = jnp.concatenate(outs, axis=-1)
    se2 = _dot(o, mWo[...]) + mbo[...]
    hq = jnp.concatenate([se2, jnp.broadcast_to(ge, (se2.shape[0], D))], axis=1)
    hq = jax.nn.relu(_ln(_dot(hq, qW1[...]) + qb1[...], qg1[...], qbe1[...]))
    hq = jax.nn.relu(_ln(_dot(hq, qW2[...]) + qb2[...], qg2[...], qbe2[...]))
    qv = _dot(hq, qW3[...]) + qb3[...]
    qv_ref[...] = qv[:, 0]
    hd = jax.nn.relu(_ln(_dot(ge, dW1[...]) + db1[...], dg1[...], dbe1[...]))
    dl = _dot(hd, dW2[...], hi=True) + db2[...]
    dl_ref[...] = dl[:, 0]


def _heads_call(sheet_embs, g_emb, sheet_features, params):
    gp, mp, qp, dp = params['geo'], params['mha'], params['qmlp'], params['done']
    args = [sheet_embs, g_emb, sheet_features,
            gp['W1'], gp['b1'], gp['g1'], gp['be1'],
            gp['W2'], gp['b2'], gp['g2'], gp['be2'],
            mp['Wq'], mp['bq'], mp['Wk'], mp['bk'],
            mp['Wv'], mp['bv'], mp['Wo'], mp['bo'],
            qp['W1'], qp['b1'], qp['g1'], qp['be1'],
            qp['W2'], qp['b2'], qp['g2'], qp['be2'], qp['W3'], qp['b3'],
            dp['W1'], dp['b1'], dp['g1'], dp['be1'], dp['W2'], dp['b2']]
    args = [a.reshape(1, -1) if a.ndim == 1 else a for a in args]
    qv, dl = pl.pallas_call(
        _heads_body,
        out_shape=(jax.ShapeDtypeStruct((256,), jnp.float32),
                   jax.ShapeDtypeStruct((1,), jnp.float32)),
    )(*args)
    return qv, dl


def _gat_layer_jnp(h, src, dst, N, p):
    xl = (h @ p['Wl'] + p['bl']).reshape(N, HEADS, CH)
    xr = (h @ p['Wr'] + p['br']).reshape(N, HEADS, CH)
    m = xl[src] + xr[dst]
    e = (jax.nn.leaky_relu(m, 0.2) * p['att']).sum(-1)
    amax = jax.ops.segment_max(e, dst, num_segments=N)
    ex = jnp.exp(e - amax[dst])
    den = jax.ops.segment_sum(ex, dst, num_segments=N)
    alpha = ex / (den[dst] + 1e-16)
    out = jax.ops.segment_sum(xl[src] * alpha[:, :, None], dst, num_segments=N)
    return out.reshape(N, D) + p['bias']


def kernel(x, edge_index, batch, sheet_node_idx, sheet_features, params):
    N = x.shape[0]
    h = x @ params['pre_W'] + params['pre_b']
    loop = jnp.arange(N, dtype=edge_index.dtype)
    src = jnp.concatenate([edge_index[0], loop])
    dst = jnp.concatenate([edge_index[1], loop])
    for p in params['gat']:
        h = jax.nn.elu(_gat_layer_jnp(h, src, dst, N, p))
    g_emb = h.mean(axis=0, keepdims=True)
    sheet_embs = h[sheet_node_idx].mean(axis=1)
    q_values, state_done_logit = _heads_call(
        sheet_embs, g_emb, sheet_features, params)
    return q_values, state_done_logit


# speedup vs baseline: 20.9412x; 20.9200x over previous
import functools

import jax
import jax.numpy as jnp
import numpy as np
from jax import lax
from jax.experimental import pallas as pl
from jax.experimental.pallas import tpu as pltpu
from jax.experimental.pallas import tpu_sc as plsc

HEADS = 4
CH = 16
D = 64
F = 10
N = 50000
E = 800000

EC = 50176
E_PAD = EC * 16
B = 512
NCHUNK = EC // B
NP = 50048
NR = NP // 16
DW = 6256
DENW = DW * 16
RB = 1000
XR_W = 48


def _dot(a, b, hi=False):
    return jax.lax.dot_general(
        a, b, (((1,), (0,)), ((), ())),
        precision=jax.lax.Precision.HIGHEST if hi else None,
        preferred_element_type=jnp.float32)


def _elu(x):
    return jnp.where(x > 0, x, jnp.exp(x) - 1.0)


def _emit_xlxr(h, Wl, bl, Wr, br, att_flat, xl0_ref, xl1_ref, xr0_ref, xr1_ref):
    xl = _dot(h, Wl[...]) + bl[...]
    xr = _dot(h, Wr[...]) + br[...]
    m = xl + xr
    lr = jnp.maximum(m, 0.2 * m)
    prod = lr * att_flat[...]
    es = [jnp.sum(prod[:, hh * CH:(hh + 1) * CH], axis=1, keepdims=True)
          for hh in range(HEADS)]
    z = jnp.zeros((h.shape[0], XR_W - 34), jnp.float32)
    xl0_ref[...] = xl[:, :32]
    xl1_ref[...] = xl[:, 32:]
    xr0_ref[...] = jnp.concatenate([xr[:, :32], es[0], es[1], z], axis=1)
    xr1_ref[...] = jnp.concatenate([xr[:, 32:], es[2], es[3], z], axis=1)


def _prep1_body(x_ref, preW, preb, Wl, bl, Wr, br, att_flat,
                xl0_ref, xl1_ref, xr0_ref, xr1_ref):
    h = _dot(x_ref[...], preW[...]) + preb[...]
    _emit_xlxr(h, Wl, bl, Wr, br, att_flat, xl0_ref, xl1_ref, xr0_ref, xr1_ref)


def _next_h(o0, o1, d0, d1, xp0, xp1, bias):
    a0 = 1.0 / (d0[...] + 1.0)
    a1 = 1.0 / (d1[...] + 1.0)
    r0 = jnp.concatenate([jnp.broadcast_to(a0[:, 0:1], (a0.shape[0], CH)),
                          jnp.broadcast_to(a0[:, 1:2], (a0.shape[0], CH))], axis=1)
    r1 = jnp.concatenate([jnp.broadcast_to(a1[:, 0:1], (a1.shape[0], CH)),
                          jnp.broadcast_to(a1[:, 1:2], (a1.shape[0], CH))], axis=1)
    out = jnp.concatenate([(o0[...] + xp0[...]) * r0,
                           (o1[...] + xp1[...]) * r1], axis=1)
    return _elu(out + bias[...])


def _prep23_body(o0, o1, d0, d1, xp0, xp1, bias, Wl, bl, Wr, br, att_flat,
                 xl0_ref, xl1_ref, xr0_ref, xr1_ref):
    h = _next_h(o0, o1, d0, d1, xp0, xp1, bias)
    _emit_xlxr(h, Wl, bl, Wr, br, att_flat, xl0_ref, xl1_ref, xr0_ref, xr1_ref)


def _final_body(o0, o1, d0, d1, xp0, xp1, bias, h_ref, gsum_ref):
    h = _next_h(o0, o1, d0, d1, xp0, xp1, bias)
    h_ref[...] = h

    @pl.when(pl.program_id(0) == 0)
    def _():
        gsum_ref[...] = jnp.zeros_like(gsum_ref)

    gsum_ref[...] += jnp.sum(h, axis=0, keepdims=True)


_row_spec = lambda w: pl.BlockSpec((RB, w), lambda i: (i, 0))
_rep_spec = lambda s: pl.BlockSpec(s, lambda i: (0, 0))


def _prep1(x, pre_W, pre_b, p):
    return pl.pallas_call(
        _prep1_body,
        grid=(N // RB,),
        in_specs=[_row_spec(F), _rep_spec((F, D)), _rep_spec((1, D)),
                  _rep_spec((D, D)), _rep_spec((1, D)),
                  _rep_spec((D, D)), _rep_spec((1, D)), _rep_spec((1, D))],
        out_specs=[_row_spec(32), _row_spec(32), _row_spec(XR_W), _row_spec(XR_W)],
        out_shape=[jax.ShapeDtypeStruct((N, 32), jnp.float32),
                   jax.ShapeDtypeStruct((N, 32), jnp.float32),
                   jax.ShapeDtypeStruct((N, XR_W), jnp.float32),
                   jax.ShapeDtypeStruct((N, XR_W), jnp.float32)],
    )(x, pre_W, pre_b.reshape(1, D), p['Wl'], p['bl'].reshape(1, D),
      p['Wr'], p['br'].reshape(1, D), p['att'].reshape(1, D))


def _prep23(o0, o1, d0, d1, xp0, xp1, bias, p):
    return pl.pallas_call(
        _prep23_body,
        grid=(N // RB,),
        in_specs=[_row_spec(32), _row_spec(32), _row_spec(2), _row_spec(2),
                  _row_spec(32), _row_spec(32), _rep_spec((1, D)),
                  _rep_spec((D, D)), _rep_spec((1, D)),
                  _rep_spec((D, D)), _rep_spec((1, D)), _rep_spec((1, D))],
        out_specs=[_row_spec(32), _row_spec(32), _row_spec(XR_W), _row_spec(XR_W)],
        out_shape=[jax.ShapeDtypeStruct((N, 32), jnp.float32),
                   jax.ShapeDtypeStruct((N, 32), jnp.float32),
                   jax.ShapeDtypeStruct((N, XR_W), jnp.float32),
                   jax.ShapeDtypeStruct((N, XR_W), jnp.float32)],
    )(o0, o1, d0, d1, xp0, xp1, bias.reshape(1, D),
      p['Wl'], p['bl'].reshape(1, D), p['Wr'], p['br'].reshape(1, D),
      p['att'].reshape(1, D))


def _final(o0, o1, d0, d1, xp0, xp1, bias):
    return pl.pallas_call(
        _final_body,
        grid=(N // RB,),
        in_specs=[_row_spec(32), _row_spec(32), _row_spec(2), _row_spec(2),
                  _row_spec(32), _row_spec(32), _rep_spec((1, D))],
        out_specs=[_row_spec(D), pl.BlockSpec((1, D), lambda i: (0, 0))],
        out_shape=[jax.ShapeDtypeStruct((N, D), jnp.float32),
                   jax.ShapeDtypeStruct((1, D), jnp.float32)],
    )(o0, o1, d0, d1, xp0, xp1, bias.reshape(1, D))


def _idx_rows(i, s):
    return s * (EC // 128) + i * 8


def _sca_body(src2, dst2, xl_all, xr_all, xlg, xrg,
              src_g, dst_go, xlb, xrb, sem_a, sem_b):
    c = lax.axis_index("c")
    s = lax.axis_index("s")
    cN = c * N

    def _p(i, car):
        row0 = _idx_rows(i, s)
        pltpu.sync_copy(src2.at[pl.ds(row0, 8), :], src_g)
        pltpu.sync_copy(dst2.at[pl.ds(row0, 8), :], dst_go)

        def _off(k, car2):
            r = k // 8
            col = (k % 8) * 16
            src_g[r, pl.ds(col, 16)] = src_g[r, pl.ds(col, 16)] + cN
            dst_go[r, pl.ds(col, 16)] = dst_go[r, pl.ds(col, 16)] + cN
            return car2
        lax.fori_loop(0, 64, _off, 0)

        for half in range(2):
            cl = []
            for g in range(4):
                cl.append(pltpu.async_copy(
                    xl_all.at[src_g.at[half * 4 + g]],
                    xlb.at[pl.ds(g * 128, 128), :], sem_a))
                cl.append(pltpu.async_copy(
                    xr_all.at[dst_go.at[half * 4 + g]],
                    xrb.at[pl.ds(g * 128, 128), :], sem_b))
            for cp in cl:
                cp.wait()
            cbase = s * EC + i * 1024 + half * 512
            pltpu.sync_copy(xlb, xlg.at[c, pl.ds(cbase, 512), :])
            pltpu.sync_copy(xrb, xrg.at[c, pl.ds(cbase, 512), :])
        return car
    lax.fori_loop(0, NCHUNK // 2, _p, 0)


def _sc_gather(src2, dst2, xl_all, xr_all):
    mesh = plsc.VectorSubcoreMesh(core_axis_name="c", subcore_axis_name="s")
    f = pl.kernel(
        _sca_body,
        compiler_params=pltpu.CompilerParams(use_tc_tiling_on_sc=False),
        out_type=(jax.ShapeDtypeStruct((2, E_PAD, 32), jnp.float32),
                  jax.ShapeDtypeStruct((2, E_PAD, XR_W), jnp.float32)),
        mesh=mesh,
        scratch_types=[
            pltpu.VMEM((8, 128), jnp.int32),
            pltpu.VMEM((8, 128), jnp.int32),
            pltpu.VMEM((B, 32), jnp.float32),
            pltpu.VMEM((B, XR_W), jnp.float32),
            pltpu.SemaphoreType.DMA,
            pltpu.SemaphoreType.DMA,
        ],
    )
    return f(src2, dst2, xl_all, xr_all)


EB = 2048
NEB = 2 * E_PAD // EB


def _tce_body(xl_ref, xr_ref, att_ref, ex0_ref, ex1_ref, w_ref):
    pid = pl.program_id(0)
    xlv = xl_ref[...]
    xrv = xr_ref[...]
    att = att_ref[...]
    is1 = pid >= (E_PAD // EB)
    attrow = jnp.where(is1, att[1:2, :], att[0:1, :])
    m = xlv + xrv[:, :32]
    lr = jnp.maximum(m, 0.2 * m)
    prod = lr * attrow
    e0 = jnp.sum(prod[:, :CH], axis=1, keepdims=True)
    e1 = jnp.sum(prod[:, CH:32], axis=1, keepdims=True)
    ridx = lax.broadcasted_iota(jnp.int32, (EB, 1), 0) + pid * EB
    edge = ridx - jnp.where(is1, E_PAD, 0)
    mask = edge < E
    ex0 = jnp.where(mask, jnp.exp(e0 - xrv[:, 32:33]), 0.0)
    ex1 = jnp.where(mask, jnp.exp(e1 - xrv[:, 33:34]), 0.0)
    ex0_ref[...] = ex0
    ex1_ref[...] = ex1
    w_ref[...] = xlv * jnp.concatenate(
        [jnp.broadcast_to(ex0, (EB, CH)), jnp.broadcast_to(ex1, (EB, CH))],
        axis=1)


def _tc_edge(xlg, xrg, att32):
    xl2 = xlg.reshape(2 * E_PAD, 32)
    xr2 = xrg.reshape(2 * E_PAD, XR_W)
    return pl.pallas_call(
        _tce_body,
        grid=(NEB,),
        in_specs=[pl.BlockSpec((EB, 32), lambda i: (i, 0)),
                  pl.BlockSpec((EB, XR_W), lambda i: (i, 0)),
                  pl.BlockSpec((2, 32), lambda i: (0, 0))],
        out_specs=[pl.BlockSpec((EB, 1), lambda i: (i, 0)),
                   pl.BlockSpec((EB, 1), lambda i: (i, 0)),
                   pl.BlockSpec((EB, 32), lambda i: (i, 0))],
        out_shape=[jax.ShapeDtypeStruct((2 * E_PAD, 1), jnp.float32),
                   jax.ShapeDtypeStruct((2 * E_PAD, 1), jnp.float32),
                   jax.ShapeDtypeStruct((2 * E_PAD, 32), jnp.float32)],
    )(xl2, xr2, att32)


def _scb_body(dst2, ex0, ex1, w,
              out_hbm, den_hbm,
              dst_g, idn, exb, wb, wbuf, sp_out, sp_den, sem_a, sem_b):
    c = lax.axis_index("c")
    s = lax.axis_index("s")
    zero16 = jnp.zeros((16,), jnp.float32)

    def _z(k, car):
        wbuf[pl.ds(k * 16, 16)] = zero16
        return car
    lax.fori_loop(0, 64, _z, 0)

    def _zw(j, car):
        wb[j, pl.ds(0, 16)] = zero16
        wb[j, pl.ds(16, 16)] = zero16
        return car
    lax.fori_loop(0, B, _zw, 0)

    r0 = s * NR
    for k in range(6):
        pltpu.sync_copy(wb, sp_out.at[pl.ds(r0 + k * 512, 512), :])
    pltpu.sync_copy(wb.at[pl.ds(0, NR - 3072), :],
                    sp_out.at[pl.ds(r0 + 3072, NR - 3072), :])
    w0 = s * DW
    for k in range(6):
        pltpu.sync_copy(wbuf, sp_den.at[pl.ds(w0 + k * 1024, 1024)])
    pltpu.sync_copy(wbuf.at[pl.ds(0, DW - 6144)],
                    sp_den.at[pl.ds(w0 + 6144, DW - 6144)])
    plsc.subcore_barrier()

    def _p(i, car):
        row0 = _idx_rows(i, s)
        pltpu.sync_copy(dst2.at[pl.ds(row0, 8), :], dst_g)
        for half in range(2):
            def _bi(k, car2):
                r = half * 4 + k // 8
                col = (k % 8) * 16
                d16 = dst_g[r, pl.ds(col, 16)]
                for hh in range(2):
                    p0 = hh * B + k * 16
                    idn[p0 // 128, pl.ds(p0 % 128, 16)] = d16 * 2 + hh
                return car2
            lax.fori_loop(0, 32, _bi, 0)

            cbase = s * EC + i * 1024 + half * 512
            cl = [pltpu.async_copy(ex0.at[pl.ds(c * E_PAD + cbase, 512)],
                                   exb.at[pl.ds(0, 512)], sem_b),
                  pltpu.async_copy(ex1.at[pl.ds(c * E_PAD + cbase, 512)],
                                   exb.at[pl.ds(512, 512)], sem_b),
                  pltpu.async_copy(w.at[c, pl.ds(cbase, 512), :], wb, sem_a)]
            for cp in cl:
                cp.wait()
            for g in range(8):
                pltpu.sync_copy(exb.at[pl.ds(g * 128, 128)],
                                sp_den.at[idn.at[g]], add=True)
            for g in range(4):
                pltpu.sync_copy(wb.at[pl.ds(g * 128, 128), :],
                                sp_out.at[dst_g.at[half * 4 + g]], add=True)
        return car
    lax.fori_loop(0, NCHUNK // 2, _p, 0)
    plsc.subcore_barrier()

    pltpu.sync_copy(sp_out.at[pl.ds(r0, NR), :],
                    out_hbm.at[c, pl.ds(r0, NR), :])
    pltpu.sync_copy(sp_den.at[pl.ds(w0, DW)],
                    den_hbm.at[pl.ds(c * DENW + w0, DW)])


def _sc_scatter(dst2, ex0, ex1, w):
    mesh = plsc.VectorSubcoreMesh(core_axis_name="c", subcore_axis_name="s")
    f = pl.kernel(
        _scb_body,
        compiler_params=pltpu.CompilerParams(use_tc_tiling_on_sc=False),
        out_type=(jax.ShapeDtypeStruct((2, NP, 32), jnp.float32),
                  jax.ShapeDtypeStruct((2 * DENW,), jnp.float32)),
        mesh=mesh,
        scratch_types=[
            pltpu.VMEM((8, 128), jnp.int32),
            pltpu.VMEM((8, 128), jnp.int32),
            pltpu.VMEM((2 * B,), jnp.float32),
            pltpu.VMEM((B, 32), jnp.float32),
            pltpu.VMEM((1024,), jnp.float32),
            pltpu.VMEM_SHARED((NP, 32), jnp.float32),
            pltpu.VMEM_SHARED((DENW,), jnp.float32),
            pltpu.SemaphoreType.DMA,
            pltpu.SemaphoreType.DMA,
        ],
    )
    return f(dst2, ex0, ex1, w)


def _sheet_body(h3, sni, ssum, idxb, rows, accb, sem):
    c = lax.axis_index("c")
    s = lax.axis_index("s")
    wid = s * 2 + c

    def _one(sh, car):
        sid = wid * 8 + sh
        pltpu.sync_copy(sni.at[sid], idxb)
        cl = []
        for g in range(4):
            cl.append(pltpu.async_copy(
                h3.at[idxb.at[g]], rows.at[pl.ds(g * 128, 128), :], sem))
        for cp in cl:
            cp.wait()

        def _acc(r, carry):
            return tuple(carry[q] + rows[r, pl.ds(q * 16, 16)]
                         for q in range(4))
        acc = lax.fori_loop(0, 512, _acc,
                            tuple(jnp.zeros((16,), jnp.float32)
                                  for _ in range(4)))
        for q in range(4):
            accb[pl.ds(q * 16, 16)] = acc[q]
        pltpu.sync_copy(accb, ssum.at[sid])
        return car
    lax.fori_loop(0, 8, _one, 0)


def _sc_sheet(h3, sni):
    mesh = plsc.VectorSubcoreMesh(core_axis_name="c", subcore_axis_name="s")
    f = pl.kernel(
        _sheet_body,
        compiler_params=pltpu.CompilerParams(use_tc_tiling_on_sc=False),
        out_type=jax.ShapeDtypeStruct((256, D), jnp.float32),
        mesh=mesh,
        scratch_types=[
            pltpu.VMEM((4, 128), jnp.int32),
            pltpu.VMEM((512, D), jnp.float32),
            pltpu.VMEM((D,), jnp.float32),
            pltpu.SemaphoreType.DMA,
        ],
    )
    return f(h3, sni)


def _ln(x, g, b):
    mu = x.mean(-1, keepdims=True)
    var = x.var(-1, keepdims=True)
    return (x - mu) / jnp.sqrt(var + 1e-5) * g + b


def _heads_body(ss_ref, gs_ref, sf_ref,
                gW1, gb1, gg1, gbe1, gW2, gb2, gg2, gbe2,
                mWq, mbq, mWk, mbk, mWv, mbv, mWo, mbo,
                qW1, qb1, qg1, qbe1, qW2, qb2, qg2, qbe2, qW3, qb3,
                dW1, db1, dg1, dbe1, dW2, db2,
                qv_ref, dl_ref):
    se = ss_ref[...] / 512.0
    ge = gs_ref[...] / float(N)
    sf = sf_ref[...]
    geo = jax.nn.relu(_ln(_dot(sf, gW1[...]) + gb1[...], gg1[...], gbe1[...]))
    geo = jax.nn.relu(_ln(_dot(geo, gW2[...]) + gb2[...], gg2[...], gbe2[...]))
    qh = _dot(se, mWq[...]) + mbq[...]
    kh = _dot(geo, mWk[...]) + mbk[...]
    vh = _dot(geo, mWv[...]) + mbv[...]
    outs = []
    scale = 1.0 / np.sqrt(CH)
    for h in range(HEADS):
        q1 = qh[:, h * CH:(h + 1) * CH]
        k1 = kh[:, h * CH:(h + 1) * CH]
        v1 = vh[:, h * CH:(h + 1) * CH]
        att = _dot(q1, k1.T) * scale
        att = jax.nn.softmax(att, axis=-1)
        outs.append(_dot(att, v1))
    o = jnp.concatenate(outs, axis=-1)
    se2 = _dot(o, mWo[...]) + mbo[...]
    hq = jnp.concatenate([se2, jnp.broadcast_to(ge, (se2.shape[0], D))], axis=1)
    hq = jax.nn.relu(_ln(_dot(hq, qW1[...]) + qb1[...], qg1[...], qbe1[...]))
    hq = jax.nn.relu(_ln(_dot(hq, qW2[...]) + qb2[...], qg2[...], qbe2[...]))
    qv = _dot(hq, qW3[...]) + qb3[...]
    qv_ref[...] = qv[:, 0]
    hd = jax.nn.relu(_ln(_dot(ge, dW1[...]) + db1[...], dg1[...], dbe1[...]))
    dl = _dot(hd, dW2[...], hi=True) + db2[...]
    dl_ref[...] = dl[:, 0]


def _heads_call(sheet_sum, g_sum, sheet_features, params):
    gp, mp, qp, dp = params['geo'], params['mha'], params['qmlp'], params['done']
    args = [sheet_sum, g_sum, sheet_features,
            gp['W1'], gp['b1'], gp['g1'], gp['be1'],
            gp['W2'], gp['b2'], gp['g2'], gp['be2'],
            mp['Wq'], mp['bq'], mp['Wk'], mp['bk'],
            mp['Wv'], mp['bv'], mp['Wo'], mp['bo'],
            qp['W1'], qp['b1'], qp['g1'], qp['be1'],
            qp['W2'], qp['b2'], qp['g2'], qp['be2'], qp['W3'], qp['b3'],
            dp['W1'], dp['b1'], dp['g1'], dp['be1'], dp['W2'], dp['b2']]
    args = [a.reshape(1, -1) if a.ndim == 1 else a for a in args]
    qv, dl = pl.pallas_call(
        _heads_body,
        out_shape=(jax.ShapeDtypeStruct((256,), jnp.float32),
                   jax.ShapeDtypeStruct((1,), jnp.float32)),
    )(*args)
    return qv, dl


def kernel(x, edge_index, batch, sheet_node_idx, sheet_features, params):
    pad = jnp.zeros((E_PAD - E,), jnp.int32)
    src2 = jnp.concatenate([edge_index[0], pad]).reshape(E_PAD // 128, 128)
    dst2 = jnp.concatenate([edge_index[1], pad]).reshape(E_PAD // 128, 128)
    sni = sheet_node_idx.reshape(256, 4, 128)

    xl0, xl1, xr0, xr1 = _prep1(x, params['pre_W'], params['pre_b'],
                                params['gat'][0])
    h3 = gsum = None
    for l in range(3):
        p = params['gat'][l]
        attp = p['att'].reshape(2, 32)
        xl_all = jnp.concatenate([xl0, xl1], axis=0)
        xr_all = jnp.concatenate([xr0, xr1], axis=0)
        xlg, xrg = _sc_gather(src2, dst2, xl_all, xr_all)
        ex0, ex1, w = _tc_edge(xlg, xrg, attp)
        out2, den = _sc_scatter(dst2, ex0.reshape(-1), ex1.reshape(-1),
                                w.reshape(2, E_PAD, 32))
        o0 = out2[0, :N]
        o1 = out2[1, :N]
        d0 = den[:DENW].reshape(DENW // 2, 2)[:N]
        d1 = den[DENW:].reshape(DENW // 2, 2)[:N]
        if l < 2:
            xl0, xl1, xr0, xr1 = _prep23(
                o0, o1, d0, d1, xl0, xl1, p['bias'],
                params['gat'][l + 1])
        else:
            h3, gsum = _final(o0, o1, d0, d1, xl0, xl1, p['bias'])

    ssum = _sc_sheet(h3, sni)
    q_values, state_done_logit = _heads_call(
        ssum, gsum, sheet_features, params)
    return q_values, state_done_logit


# speedup vs baseline: 22.0953x; 1.0551x over previous
import functools

import jax
import jax.numpy as jnp
import numpy as np
from jax import lax
from jax.experimental import pallas as pl
from jax.experimental.pallas import tpu as pltpu
from jax.experimental.pallas import tpu_sc as plsc

HEADS = 4
CH = 16
D = 64
F = 10
N = 50000
E = 800000

EC = 50176
E_PAD = EC * 16
B = 512
NCHUNK = EC // B
NP = 50048
NR = NP // 16
DW = 6256
DENW = DW * 16
RB = 1000
XR_W = 48
WX = 32
DX = 8


def _dot(a, b, hi=False):
    return jax.lax.dot_general(
        a, b, (((1,), (0,)), ((), ())),
        precision=jax.lax.Precision.HIGHEST if hi else None,
        preferred_element_type=jnp.float32)


def _elu(x):
    return jnp.where(x > 0, x, jnp.exp(x) - 1.0)


def _emit_xlxr(h, Wl, bl, Wr, br, att_flat, xl0_ref, xl1_ref, xr0_ref, xr1_ref):
    xl = _dot(h, Wl[...]) + bl[...]
    xr = _dot(h, Wr[...]) + br[...]
    m = xl + xr
    lr = jnp.maximum(m, 0.2 * m)
    prod = lr * att_flat[...]
    es = [jnp.sum(prod[:, hh * CH:(hh + 1) * CH], axis=1, keepdims=True)
          for hh in range(HEADS)]
    z = jnp.zeros((h.shape[0], XR_W - 34), jnp.float32)
    xl0_ref[...] = xl[:, :32]
    xl1_ref[...] = xl[:, 32:]
    xr0_ref[...] = jnp.concatenate([xr[:, :32], es[0], es[1], z], axis=1)
    xr1_ref[...] = jnp.concatenate([xr[:, 32:], es[2], es[3], z], axis=1)


def _prep1_body(x_ref, preW, preb, Wl, bl, Wr, br, att_flat,
                xl0_ref, xl1_ref, xr0_ref, xr1_ref):
    h = _dot(x_ref[...], preW[...]) + preb[...]
    _emit_xlxr(h, Wl, bl, Wr, br, att_flat, xl0_ref, xl1_ref, xr0_ref, xr1_ref)


def _next_h(o0, o1, d0, d1, xp0, xp1, bias):
    a0 = 1.0 / (d0[...] + 1.0)
    a1 = 1.0 / (d1[...] + 1.0)
    r0 = jnp.concatenate([jnp.broadcast_to(a0[:, 0:1], (a0.shape[0], CH)),
                          jnp.broadcast_to(a0[:, 1:2], (a0.shape[0], CH))], axis=1)
    r1 = jnp.concatenate([jnp.broadcast_to(a1[:, 0:1], (a1.shape[0], CH)),
                          jnp.broadcast_to(a1[:, 1:2], (a1.shape[0], CH))], axis=1)
    out = jnp.concatenate([(o0[...] + xp0[...]) * r0,
                           (o1[...] + xp1[...]) * r1], axis=1)
    return _elu(out + bias[...])


def _prep23_body(o0, o1, d0, d1, xp0, xp1, bias, Wl, bl, Wr, br, att_flat,
                 xl0_ref, xl1_ref, xr0_ref, xr1_ref):
    h = _next_h(o0, o1, d0, d1, xp0, xp1, bias)
    _emit_xlxr(h, Wl, bl, Wr, br, att_flat, xl0_ref, xl1_ref, xr0_ref, xr1_ref)


def _final_body(o0, o1, d0, d1, xp0, xp1, bias, h_ref, gsum_ref):
    h = _next_h(o0, o1, d0, d1, xp0, xp1, bias)
    h_ref[...] = h

    @pl.when(pl.program_id(0) == 0)
    def _():
        gsum_ref[...] = jnp.zeros_like(gsum_ref)

    gsum_ref[...] += jnp.sum(h, axis=0, keepdims=True)


_row_spec = lambda w: pl.BlockSpec((RB, w), lambda i: (i, 0))
_rep_spec = lambda s: pl.BlockSpec(s, lambda i: (0, 0))


def _prep1(x, pre_W, pre_b, p):
    return pl.pallas_call(
        _prep1_body,
        grid=(N // RB,),
        in_specs=[_row_spec(F), _rep_spec((F, D)), _rep_spec((1, D)),
                  _rep_spec((D, D)), _rep_spec((1, D)),
                  _rep_spec((D, D)), _rep_spec((1, D)), _rep_spec((1, D))],
        out_specs=[_row_spec(32), _row_spec(32), _row_spec(XR_W), _row_spec(XR_W)],
        out_shape=[jax.ShapeDtypeStruct((N, 32), jnp.float32),
                   jax.ShapeDtypeStruct((N, 32), jnp.float32),
                   jax.ShapeDtypeStruct((N, XR_W), jnp.float32),
                   jax.ShapeDtypeStruct((N, XR_W), jnp.float32)],
    )(x, pre_W, pre_b.reshape(1, D), p['Wl'], p['bl'].reshape(1, D),
      p['Wr'], p['br'].reshape(1, D), p['att'].reshape(1, D))


def _prep23(o0, o1, d0, d1, xp0, xp1, bias, p):
    return pl.pallas_call(
        _prep23_body,
        grid=(N // RB,),
        in_specs=[_row_spec(32), _row_spec(32), _row_spec(2), _row_spec(2),
                  _row_spec(32), _row_spec(32), _rep_spec((1, D)),
                  _rep_spec((D, D)), _rep_spec((1, D)),
                  _rep_spec((D, D)), _rep_spec((1, D)), _rep_spec((1, D))],
        out_specs=[_row_spec(32), _row_spec(32), _row_spec(XR_W), _row_spec(XR_W)],
        out_shape=[jax.ShapeDtypeStruct((N, 32), jnp.float32),
                   jax.ShapeDtypeStruct((N, 32), jnp.float32),
                   jax.ShapeDtypeStruct((N, XR_W), jnp.float32),
                   jax.ShapeDtypeStruct((N, XR_W), jnp.float32)],
    )(o0, o1, d0, d1, xp0, xp1, bias.reshape(1, D),
      p['Wl'], p['bl'].reshape(1, D), p['Wr'], p['br'].reshape(1, D),
      p['att'].reshape(1, D))


def _final(o0, o1, d0, d1, xp0, xp1, bias):
    return pl.pallas_call(
        _final_body,
        grid=(N // RB,),
        in_specs=[_row_spec(32), _row_spec(32), _row_spec(2), _row_spec(2),
                  _row_spec(32), _row_spec(32), _rep_spec((1, D))],
        out_specs=[_row_spec(D), pl.BlockSpec((1, D), lambda i: (0, 0))],
        out_shape=[jax.ShapeDtypeStruct((N, D), jnp.float32),
                   jax.ShapeDtypeStruct((1, D), jnp.float32)],
    )(o0, o1, d0, d1, xp0, xp1, bias.reshape(1, D))


def _idx_rows(i, s):
    return s * (EC // 128) + i * 8


def _sca_body(src2, dst2, xl_all, xr_all, xlg, xrg,
              src_g, dst_go, xlb, xrb, sem_a, sem_b):
    c = lax.axis_index("c")
    s = lax.axis_index("s")
    cN = c * N

    def _p(i, car):
        row0 = _idx_rows(i, s)
        pltpu.sync_copy(src2.at[pl.ds(row0, 8), :], src_g)
        pltpu.sync_copy(dst2.at[pl.ds(row0, 8), :], dst_go)

        def _off(k, car2):
            r = k // 8
            col = (k % 8) * 16
            src_g[r, pl.ds(col, 16)] = src_g[r, pl.ds(col, 16)] + cN
            dst_go[r, pl.ds(col, 16)] = dst_go[r, pl.ds(col, 16)] + cN
            return car2
        lax.fori_loop(0, 64, _off, 0)

        for half in range(2):
            cl = []
            for g in range(4):
                cl.append(pltpu.async_copy(
                    xl_all.at[src_g.at[half * 4 + g]],
                    xlb.at[pl.ds(g * 128, 128), :], sem_a))
                cl.append(pltpu.async_copy(
                    xr_all.at[dst_go.at[half * 4 + g]],
                    xrb.at[pl.ds(g * 128, 128), :], sem_b))
            for cp in cl:
                cp.wait()
            cbase = s * EC + i * 1024 + half * 512
            pltpu.sync_copy(xlb, xlg.at[c, pl.ds(cbase, 512), :])
            pltpu.sync_copy(xrb, xrg.at[c, pl.ds(cbase, 512), :])
        return car
    lax.fori_loop(0, NCHUNK // 2, _p, 0)


def _sc_gather(src2, dst2, xl_all, xr_all):
    mesh = plsc.VectorSubcoreMesh(core_axis_name="c", subcore_axis_name="s")
    f = pl.kernel(
        _sca_body,
        compiler_params=pltpu.CompilerParams(use_tc_tiling_on_sc=False),
        out_type=(jax.ShapeDtypeStruct((2, E_PAD, 32), jnp.float32),
                  jax.ShapeDtypeStruct((2, E_PAD, XR_W), jnp.float32)),
        mesh=mesh,
        scratch_types=[
            pltpu.VMEM((8, 128), jnp.int32),
            pltpu.VMEM((8, 128), jnp.int32),
            pltpu.VMEM((B, 32), jnp.float32),
            pltpu.VMEM((B, XR_W), jnp.float32),
            pltpu.SemaphoreType.DMA,
            pltpu.SemaphoreType.DMA,
        ],
    )
    return f(src2, dst2, xl_all, xr_all)


EB = 2048
NEB = 2 * E_PAD // EB


def _tce_body(xl_ref, xr_ref, att_ref, w_ref, exw_ref):
    pid = pl.program_id(0)
    xlv = xl_ref[...]
    xrv = xr_ref[...]
    att = att_ref[...]
    is1 = pid >= (E_PAD // EB)
    attrow = jnp.where(is1, att[1:2, :], att[0:1, :])
    m = xlv + xrv[:, :32]
    lr = jnp.maximum(m, 0.2 * m)
    prod = lr * attrow
    e0 = jnp.sum(prod[:, :CH], axis=1, keepdims=True)
    e1 = jnp.sum(prod[:, CH:32], axis=1, keepdims=True)
    ridx = lax.broadcasted_iota(jnp.int32, (EB, 1), 0) + pid * EB
    edge = ridx - jnp.where(is1, E_PAD, 0)
    mask = edge < E
    ex0 = jnp.where(mask, jnp.exp(e0 - xrv[:, 32:33]), 0.0)
    ex1 = jnp.where(mask, jnp.exp(e1 - xrv[:, 33:34]), 0.0)
    w_ref[...] = xlv * jnp.concatenate(
        [jnp.broadcast_to(ex0, (EB, CH)), jnp.broadcast_to(ex1, (EB, CH))],
        axis=1)
    exw_ref[...] = jnp.concatenate(
        [ex0, ex1, jnp.zeros((EB, DX - 2), jnp.float32)], axis=1)


def _tc_edge(xlg, xrg, att32):
    xl2 = xlg.reshape(2 * E_PAD, 32)
    xr2 = xrg.reshape(2 * E_PAD, XR_W)
    return pl.pallas_call(
        _tce_body,
        grid=(NEB,),
        in_specs=[pl.BlockSpec((EB, 32), lambda i: (i, 0)),
                  pl.BlockSpec((EB, XR_W), lambda i: (i, 0)),
                  pl.BlockSpec((2, 32), lambda i: (0, 0))],
        out_specs=[pl.BlockSpec((EB, WX), lambda i: (i, 0)),
                   pl.BlockSpec((EB, DX), lambda i: (i, 0))],
        out_shape=[jax.ShapeDtypeStruct((2 * E_PAD, WX), jnp.float32),
                   jax.ShapeDtypeStruct((2 * E_PAD, DX), jnp.float32)],
    )(xl2, xr2, att32)


def _make_scb_body(width):
    ng = width // 16

    def _scb_body(dst2, rows_in, out_hbm, dst_g, wb, sp_acc, sem_a):
        c = lax.axis_index("c")
        s = lax.axis_index("s")
        zero16 = jnp.zeros((16,), jnp.float32)
        zerow = jnp.zeros((width if width < 16 else 16,), jnp.float32)

        def _zw(j, car):
            if width >= 16:
                for q in range(ng):
                    wb[j, pl.ds(q * 16, 16)] = zero16
            else:
                wb[j, pl.ds(0, width)] = zerow
            return car
        lax.fori_loop(0, B, _zw, 0)

        r0 = s * NR
        for k in range(6):
            pltpu.sync_copy(wb, sp_acc.at[pl.ds(r0 + k * 512, 512), :])
        pltpu.sync_copy(wb.at[pl.ds(0, NR - 3072), :],
                        sp_acc.at[pl.ds(r0 + 3072, NR - 3072), :])
        plsc.subcore_barrier()

        def _p(i, car):
            row0 = _idx_rows(i, s)
            pltpu.sync_copy(dst2.at[pl.ds(row0, 8), :], dst_g)
            for half in range(2):
                cbase = s * EC + i * 1024 + half * 512
                pltpu.async_copy(rows_in.at[c, pl.ds(cbase, 512), :], wb,
                                 sem_a).wait()
                for g in range(4):
                    pltpu.sync_copy(wb.at[pl.ds(g * 128, 128), :],
                                    sp_acc.at[dst_g.at[half * 4 + g]],
                                    add=True)
            return car
        lax.fori_loop(0, NCHUNK // 2, _p, 0)
        plsc.subcore_barrier()

        pltpu.sync_copy(sp_acc.at[pl.ds(r0, NR), :],
                        out_hbm.at[c, pl.ds(r0, NR), :])
    return _scb_body


def _sc_scatter(dst2, rows_in, width):
    mesh = plsc.VectorSubcoreMesh(core_axis_name="c", subcore_axis_name="s")
    f = pl.kernel(
        _make_scb_body(width),
        compiler_params=pltpu.CompilerParams(use_tc_tiling_on_sc=False),
        out_type=jax.ShapeDtypeStruct((2, NP, width), jnp.float32),
        mesh=mesh,
        scratch_types=[
            pltpu.VMEM((8, 128), jnp.int32),
            pltpu.VMEM((B, width), jnp.float32),
            pltpu.VMEM_SHARED((NP, width), jnp.float32),
            pltpu.SemaphoreType.DMA,
        ],
    )
    return f(dst2, rows_in)


def _sheet_body(h3, sni, ssum, idxb, rows, accb, sem):
    c = lax.axis_index("c")
    s = lax.axis_index("s")
    wid = s * 2 + c

    def _one(sh, car):
        sid = wid * 8 + sh
        pltpu.sync_copy(sni.at[sid], idxb)
        cl = []
        for g in range(4):
            cl.append(pltpu.async_copy(
                h3.at[idxb.at[g]], rows.at[pl.ds(g * 128, 128), :], sem))
        for cp in cl:
            cp.wait()

        def _acc(r, carry):
            return tuple(carry[q] + rows[r, pl.ds(q * 16, 16)]
                         for q in range(4))
        acc = lax.fori_loop(0, 512, _acc,
                            tuple(jnp.zeros((16,), jnp.float32)
                                  for _ in range(4)))
        for q in range(4):
            accb[pl.ds(q * 16, 16)] = acc[q]
        pltpu.sync_copy(accb, ssum.at[sid])
        return car
    lax.fori_loop(0, 8, _one, 0)


def _sc_sheet(h3, sni):
    mesh = plsc.VectorSubcoreMesh(core_axis_name="c", subcore_axis_name="s")
    f = pl.kernel(
        _sheet_body,
        compiler_params=pltpu.CompilerParams(use_tc_tiling_on_sc=False),
        out_type=jax.ShapeDtypeStruct((256, D), jnp.float32),
        mesh=mesh,
        scratch_types=[
            pltpu.VMEM((4, 128), jnp.int32),
            pltpu.VMEM((512, D), jnp.float32),
            pltpu.VMEM((D,), jnp.float32),
            pltpu.SemaphoreType.DMA,
        ],
    )
    return f(h3, sni)


def _ln(x, g, b):
    mu = x.mean(-1, keepdims=True)
    var = x.var(-1, keepdims=True)
    return (x - mu) / jnp.sqrt(var + 1e-5) * g + b


def _heads_body(ss_ref, gs_ref, sf_ref,
                gW1, gb1, gg1, gbe1, gW2, gb2, gg2, gbe2,
                mWq, mbq, mWk, mbk, mWv, mbv, mWo, mbo,
                qW1, qb1, qg1, qbe1, qW2, qb2, qg2, qbe2, qW3, qb3,
                dW1, db1, dg1, dbe1, dW2, db2,
                qv_ref, dl_ref):
    se = ss_ref[...] / 512.0
    ge = gs_ref[...] / float(N)
    sf = sf_ref[...]
    geo = jax.nn.relu(_ln(_dot(sf, gW1[...]) + gb1[...], gg1[...], gbe1[...]))
    geo = jax.nn.relu(_ln(_dot(geo, gW2[...]) + gb2[...], gg2[...], gbe2[...]))
    qh = _dot(se, mWq[...]) + mbq[...]
    kh = _dot(geo, mWk[...]) + mbk[...]
    vh = _dot(geo, mWv[...]) + mbv[...]
    outs = []
    scale = 1.0 / np.sqrt(CH)
    for h in range(HEADS):
        q1 = qh[:, h * CH:(h + 1) * CH]
        k1 = kh[:, h * CH:(h + 1) * CH]
        v1 = vh[:, h * CH:(h + 1) * CH]
        att = _dot(q1, k1.T) * scale
        att = jax.nn.softmax(att, axis=-1)
        outs.append(_dot(att, v1))
    o = jnp.concatenate(outs, axis=-1)
    se2 = _dot(o, mWo[...]) + mbo[...]
    hq = jnp.concatenate([se2, jnp.broadcast_to(ge, (se2.shape[0], D))], axis=1)
    hq = jax.nn.relu(_ln(_dot(hq, qW1[...]) + qb1[...], qg1[...], qbe1[...]))
    hq = jax.nn.relu(_ln(_dot(hq, qW2[...]) + qb2[...], qg2[...], qbe2[...]))
    qv = _dot(hq, qW3[...]) + qb3[...]
    qv_ref[...] = qv[:, 0]
    hd = jax.nn.relu(_ln(_dot(ge, dW1[...]) + db1[...], dg1[...], dbe1[...]))
    dl = _dot(hd, dW2[...], hi=True) + db2[...]
    dl_ref[...] = dl[:, 0]


def _heads_call(sheet_sum, g_sum, sheet_features, params):
    gp, mp, qp, dp = params['geo'], params['mha'], params['qmlp'], params['done']
    args = [sheet_sum, g_sum, sheet_features,
            gp['W1'], gp['b1'], gp['g1'], gp['be1'],
            gp['W2'], gp['b2'], gp['g2'], gp['be2'],
            mp['Wq'], mp['bq'], mp['Wk'], mp['bk'],
            mp['Wv'], mp['bv'], mp['Wo'], mp['bo'],
            qp['W1'], qp['b1'], qp['g1'], qp['be1'],
            qp['W2'], qp['b2'], qp['g2'], qp['be2'], qp['W3'], qp['b3'],
            dp['W1'], dp['b1'], dp['g1'], dp['be1'], dp['W2'], dp['b2']]
    args = [a.reshape(1, -1) if a.ndim == 1 else a for a in args]
    qv, dl = pl.pallas_call(
        _heads_body,
        out_shape=(jax.ShapeDtypeStruct((256,), jnp.float32),
                   jax.ShapeDtypeStruct((1,), jnp.float32)),
    )(*args)
    return qv, dl


def kernel(x, edge_index, batch, sheet_node_idx, sheet_features, params):
    pad = jnp.zeros((E_PAD - E,), jnp.int32)
    src2 = jnp.concatenate([edge_index[0], pad]).reshape(E_PAD // 128, 128)
    dst2 = jnp.concatenate([edge_index[1], pad]).reshape(E_PAD // 128, 128)
    sni = sheet_node_idx.reshape(256, 4, 128)

    xl0, xl1, xr0, xr1 = _prep1(x, params['pre_W'], params['pre_b'],
                                params['gat'][0])
    h3 = gsum = None
    for l in range(3):
        p = params['gat'][l]
        attp = p['att'].reshape(2, 32)
        xl_all = jnp.concatenate([xl0, xl1], axis=0)
        xr_all = jnp.concatenate([xr0, xr1], axis=0)
        xlg, xrg = _sc_gather(src2, dst2, xl_all, xr_all)
        w, exw = _tc_edge(xlg, xrg, attp)
        out2 = _sc_scatter(dst2, w.reshape(2, E_PAD, WX), WX)
        den2 = _sc_scatter(dst2, exw.reshape(2, E_PAD, DX), DX)
        o0 = out2[0, :N, :]
        o1 = out2[1, :N, :]
        d0 = den2[0, :N, 0:2]
        d1 = den2[1, :N, 0:2]
        if l < 2:
            xl0, xl1, xr0, xr1 = _prep23(
                o0, o1, d0, d1, xl0, xl1, p['bias'],
                params['gat'][l + 1])
        else:
            h3, gsum = _final(o0, o1, d0, d1, xl0, xl1, p['bias'])

    ssum = _sc_sheet(h3, sni)
    q_values, state_done_logit = _heads_call(
        ssum, gsum, sheet_features, params)
    return q_values, state_done_logit


# speedup vs baseline: 22.5802x; 1.0219x over previous
import functools

import jax
import jax.numpy as jnp
import numpy as np
from jax import lax
from jax.experimental import pallas as pl
from jax.experimental.pallas import tpu as pltpu
from jax.experimental.pallas import tpu_sc as plsc

HEADS = 4
CH = 16
D = 64
F = 10
N = 50000
E = 800000

EC = 50176
E_PAD = EC * 16
B = 512
NCHUNK = EC // B
NP = 50048
NR = NP // 16
DW = 6256
DENW = DW * 16
RB = 1000
XR_W = 48
WX = 32
DX = 8


def _dot(a, b, hi=False):
    return jax.lax.dot_general(
        a, b, (((1,), (0,)), ((), ())),
        precision=jax.lax.Precision.HIGHEST if hi else None,
        preferred_element_type=jnp.float32)


def _elu(x):
    return jnp.where(x > 0, x, jnp.exp(x) - 1.0)


def _emit_xlxr(h, Wl, bl, Wr, br, att_flat, xl0_ref, xl1_ref, xr0_ref, xr1_ref):
    xl = _dot(h, Wl[...]) + bl[...]
    xr = _dot(h, Wr[...]) + br[...]
    m = xl + xr
    lr = jnp.maximum(m, 0.2 * m)
    prod = lr * att_flat[...]
    es = [jnp.sum(prod[:, hh * CH:(hh + 1) * CH], axis=1, keepdims=True)
          for hh in range(HEADS)]
    z = jnp.zeros((h.shape[0], XR_W - 34), jnp.float32)
    xl0_ref[...] = xl[:, :32]
    xl1_ref[...] = xl[:, 32:]
    xr0_ref[...] = jnp.concatenate([xr[:, :32], es[0], es[1], z], axis=1)
    xr1_ref[...] = jnp.concatenate([xr[:, 32:], es[2], es[3], z], axis=1)


def _prep1_body(x_ref, preW, preb, Wl, bl, Wr, br, att_flat,
                xl0_ref, xl1_ref, xr0_ref, xr1_ref):
    h = _dot(x_ref[...], preW[...]) + preb[...]
    _emit_xlxr(h, Wl, bl, Wr, br, att_flat, xl0_ref, xl1_ref, xr0_ref, xr1_ref)


def _next_h(o0, o1, d0, d1, xp0, xp1, bias):
    a0 = 1.0 / (d0[...] + 1.0)
    a1 = 1.0 / (d1[...] + 1.0)
    r0 = jnp.concatenate([jnp.broadcast_to(a0[:, 0:1], (a0.shape[0], CH)),
                          jnp.broadcast_to(a0[:, 1:2], (a0.shape[0], CH))], axis=1)
    r1 = jnp.concatenate([jnp.broadcast_to(a1[:, 0:1], (a1.shape[0], CH)),
                          jnp.broadcast_to(a1[:, 1:2], (a1.shape[0], CH))], axis=1)
    out = jnp.concatenate([(o0[...] + xp0[...]) * r0,
                           (o1[...] + xp1[...]) * r1], axis=1)
    return _elu(out + bias[...])


def _prep23_body(o0, o1, d0, d1, xp0, xp1, bias, Wl, bl, Wr, br, att_flat,
                 xl0_ref, xl1_ref, xr0_ref, xr1_ref):
    h = _next_h(o0, o1, d0, d1, xp0, xp1, bias)
    _emit_xlxr(h, Wl, bl, Wr, br, att_flat, xl0_ref, xl1_ref, xr0_ref, xr1_ref)


def _final_body(o0, o1, d0, d1, xp0, xp1, bias, h_ref, gsum_ref):
    h = _next_h(o0, o1, d0, d1, xp0, xp1, bias)
    h_ref[...] = h

    @pl.when(pl.program_id(0) == 0)
    def _():
        gsum_ref[...] = jnp.zeros_like(gsum_ref)

    gsum_ref[...] += jnp.sum(h, axis=0, keepdims=True)


_row_spec = lambda w: pl.BlockSpec((RB, w), lambda i: (i, 0))
_rep_spec = lambda s: pl.BlockSpec(s, lambda i: (0, 0))


def _prep1(x, pre_W, pre_b, p):
    return pl.pallas_call(
        _prep1_body,
        grid=(N // RB,),
        in_specs=[_row_spec(F), _rep_spec((F, D)), _rep_spec((1, D)),
                  _rep_spec((D, D)), _rep_spec((1, D)),
                  _rep_spec((D, D)), _rep_spec((1, D)), _rep_spec((1, D))],
        out_specs=[_row_spec(32), _row_spec(32), _row_spec(XR_W), _row_spec(XR_W)],
        out_shape=[jax.ShapeDtypeStruct((N, 32), jnp.float32),
                   jax.ShapeDtypeStruct((N, 32), jnp.float32),
                   jax.ShapeDtypeStruct((N, XR_W), jnp.float32),
                   jax.ShapeDtypeStruct((N, XR_W), jnp.float32)],
    )(x, pre_W, pre_b.reshape(1, D), p['Wl'], p['bl'].reshape(1, D),
      p['Wr'], p['br'].reshape(1, D), p['att'].reshape(1, D))


def _prep23(o0, o1, d0, d1, xp0, xp1, bias, p):
    return pl.pallas_call(
        _prep23_body,
        grid=(N // RB,),
        in_specs=[_row_spec(32), _row_spec(32), _row_spec(2), _row_spec(2),
                  _row_spec(32), _row_spec(32), _rep_spec((1, D)),
                  _rep_spec((D, D)), _rep_spec((1, D)),
                  _rep_spec((D, D)), _rep_spec((1, D)), _rep_spec((1, D))],
        out_specs=[_row_spec(32), _row_spec(32), _row_spec(XR_W), _row_spec(XR_W)],
        out_shape=[jax.ShapeDtypeStruct((N, 32), jnp.float32),
                   jax.ShapeDtypeStruct((N, 32), jnp.float32),
                   jax.ShapeDtypeStruct((N, XR_W), jnp.float32),
                   jax.ShapeDtypeStruct((N, XR_W), jnp.float32)],
    )(o0, o1, d0, d1, xp0, xp1, bias.reshape(1, D),
      p['Wl'], p['bl'].reshape(1, D), p['Wr'], p['br'].reshape(1, D),
      p['att'].reshape(1, D))


def _final(o0, o1, d0, d1, xp0, xp1, bias):
    return pl.pallas_call(
        _final_body,
        grid=(N // RB,),
        in_specs=[_row_spec(32), _row_spec(32), _row_spec(2), _row_spec(2),
                  _row_spec(32), _row_spec(32), _rep_spec((1, D))],
        out_specs=[_row_spec(D), pl.BlockSpec((1, D), lambda i: (0, 0))],
        out_shape=[jax.ShapeDtypeStruct((N, D), jnp.float32),
                   jax.ShapeDtypeStruct((1, D), jnp.float32)],
    )(o0, o1, d0, d1, xp0, xp1, bias.reshape(1, D))


def _idx_rows(i, s):
    return s * (EC // 128) + i * 8


def _sca_body(src2, dst2, xl_all, xr_all, xlg, xrg,
              src_g, dst_go, xlb, xrb, sem_a, sem_b):
    c = lax.axis_index("c")
    s = lax.axis_index("s")
    cN = c * N

    def _p(i, car):
        row0 = _idx_rows(i, s)
        pltpu.sync_copy(src2.at[pl.ds(row0, 8), :], src_g)
        pltpu.sync_copy(dst2.at[pl.ds(row0, 8), :], dst_go)

        def _off(k, car2):
            r = k // 8
            col = (k % 8) * 16
            src_g[r, pl.ds(col, 16)] = src_g[r, pl.ds(col, 16)] + cN
            dst_go[r, pl.ds(col, 16)] = dst_go[r, pl.ds(col, 16)] + cN
            return car2
        lax.fori_loop(0, 64, _off, 0)

        for half in range(2):
            cl = []
            for g in range(4):
                cl.append(pltpu.async_copy(
                    xl_all.at[src_g.at[half * 4 + g]],
                    xlb.at[pl.ds(g * 128, 128), :], sem_a))
                cl.append(pltpu.async_copy(
                    xr_all.at[dst_go.at[half * 4 + g]],
                    xrb.at[pl.ds(g * 128, 128), :], sem_b))
            for cp in cl:
                cp.wait()
            cbase = s * EC + i * 1024 + half * 512
            pltpu.sync_copy(xlb, xlg.at[c, pl.ds(cbase, 512), :])
            pltpu.sync_copy(xrb, xrg.at[c, pl.ds(cbase, 512), :])
        return car
    lax.fori_loop(0, NCHUNK // 2, _p, 0)


def _sc_gather(src2, dst2, xl_all, xr_all):
    mesh = plsc.VectorSubcoreMesh(core_axis_name="c", subcore_axis_name="s")
    f = pl.kernel(
        _sca_body,
        compiler_params=pltpu.CompilerParams(use_tc_tiling_on_sc=False),
        out_type=(jax.ShapeDtypeStruct((2, E_PAD, 32), jnp.float32),
                  jax.ShapeDtypeStruct((2, E_PAD, XR_W), jnp.float32)),
        mesh=mesh,
        scratch_types=[
            pltpu.VMEM((8, 128), jnp.int32),
            pltpu.VMEM((8, 128), jnp.int32),
            pltpu.VMEM((B, 32), jnp.float32),
            pltpu.VMEM((B, XR_W), jnp.float32),
            pltpu.SemaphoreType.DMA,
            pltpu.SemaphoreType.DMA,
        ],
    )
    return f(src2, dst2, xl_all, xr_all)


EB = 8192
NEB = 2 * E_PAD // EB


def _tce_body(xl_ref, xr_ref, att_ref, w_ref, exw_ref):
    pid = pl.program_id(0)
    xlv = xl_ref[...]
    xrv = xr_ref[...]
    att = att_ref[...]
    is1 = pid >= (E_PAD // EB)
    attrow = jnp.where(is1, att[1:2, :], att[0:1, :])
    m = xlv + xrv[:, :32]
    lr = jnp.maximum(m, 0.2 * m)
    prod = lr * attrow
    e0 = jnp.sum(prod[:, :CH], axis=1, keepdims=True)
    e1 = jnp.sum(prod[:, CH:32], axis=1, keepdims=True)
    ridx = lax.broadcasted_iota(jnp.int32, (EB, 1), 0) + pid * EB
    edge = ridx - jnp.where(is1, E_PAD, 0)
    mask = edge < E
    ex0 = jnp.where(mask, jnp.exp(e0 - xrv[:, 32:33]), 0.0)
    ex1 = jnp.where(mask, jnp.exp(e1 - xrv[:, 33:34]), 0.0)
    w_ref[...] = xlv * jnp.concatenate(
        [jnp.broadcast_to(ex0, (EB, CH)), jnp.broadcast_to(ex1, (EB, CH))],
        axis=1)
    exw_ref[...] = jnp.concatenate(
        [ex0, ex1, jnp.zeros((EB, DX - 2), jnp.float32)], axis=1)


def _tc_edge(xlg, xrg, att32):
    xl2 = xlg.reshape(2 * E_PAD, 32)
    xr2 = xrg.reshape(2 * E_PAD, XR_W)
    return pl.pallas_call(
        _tce_body,
        grid=(NEB,),
        in_specs=[pl.BlockSpec((EB, 32), lambda i: (i, 0)),
                  pl.BlockSpec((EB, XR_W), lambda i: (i, 0)),
                  pl.BlockSpec((2, 32), lambda i: (0, 0))],
        out_specs=[pl.BlockSpec((EB, WX), lambda i: (i, 0)),
                   pl.BlockSpec((EB, DX), lambda i: (i, 0))],
        out_shape=[jax.ShapeDtypeStruct((2 * E_PAD, WX), jnp.float32),
                   jax.ShapeDtypeStruct((2 * E_PAD, DX), jnp.float32)],
    )(xl2, xr2, att32)


def _make_scb_body(width):
    ng = width // 16

    def _scb_body(dst2, rows_in, out_hbm, dst_g, wb, sp_acc, sem_a):
        c = lax.axis_index("c")
        s = lax.axis_index("s")
        zero16 = jnp.zeros((16,), jnp.float32)
        zerow = jnp.zeros((width if width < 16 else 16,), jnp.float32)

        def _zw(j, car):
            if width >= 16:
                for q in range(ng):
                    wb[j, pl.ds(q * 16, 16)] = zero16
            else:
                wb[j, pl.ds(0, width)] = zerow
            return car
        lax.fori_loop(0, B, _zw, 0)

        r0 = s * NR
        for k in range(6):
            pltpu.sync_copy(wb, sp_acc.at[pl.ds(r0 + k * 512, 512), :])
        pltpu.sync_copy(wb.at[pl.ds(0, NR - 3072), :],
                        sp_acc.at[pl.ds(r0 + 3072, NR - 3072), :])
        plsc.subcore_barrier()

        def _p(i, car):
            row0 = _idx_rows(i, s)
            pltpu.sync_copy(dst2.at[pl.ds(row0, 8), :], dst_g)
            for half in range(2):
                cbase = s * EC + i * 1024 + half * 512
                pltpu.async_copy(rows_in.at[c, pl.ds(cbase, 512), :], wb,
                                 sem_a).wait()
                for g in range(4):
                    pltpu.sync_copy(wb.at[pl.ds(g * 128, 128), :],
                                    sp_acc.at[dst_g.at[half * 4 + g]],
                                    add=True)
            return car
        lax.fori_loop(0, NCHUNK // 2, _p, 0)
        plsc.subcore_barrier()

        pltpu.sync_copy(sp_acc.at[pl.ds(r0, NR), :],
                        out_hbm.at[c, pl.ds(r0, NR), :])
    return _scb_body


def _sc_scatter(dst2, rows_in, width):
    mesh = plsc.VectorSubcoreMesh(core_axis_name="c", subcore_axis_name="s")
    f = pl.kernel(
        _make_scb_body(width),
        compiler_params=pltpu.CompilerParams(use_tc_tiling_on_sc=False),
        out_type=jax.ShapeDtypeStruct((2, NP, width), jnp.float32),
        mesh=mesh,
        scratch_types=[
            pltpu.VMEM((8, 128), jnp.int32),
            pltpu.VMEM((B, width), jnp.float32),
            pltpu.VMEM_SHARED((NP, width), jnp.float32),
            pltpu.SemaphoreType.DMA,
        ],
    )
    return f(dst2, rows_in)


def _sheet_body(h3, sni, ssum, idxb, rows, accb, sem):
    c = lax.axis_index("c")
    s = lax.axis_index("s")
    wid = s * 2 + c

    def _one(sh, car):
        sid = wid * 8 + sh
        pltpu.sync_copy(sni.at[sid], idxb)
        cl = []
        for g in range(4):
            cl.append(pltpu.async_copy(
                h3.at[idxb.at[g]], rows.at[pl.ds(g * 128, 128), :], sem))
        for cp in cl:
            cp.wait()

        def _acc(r, carry):
            return tuple(carry[q] + rows[r, pl.ds(q * 16, 16)]
                         for q in range(4))
        acc = lax.fori_loop(0, 512, _acc,
                            tuple(jnp.zeros((16,), jnp.float32)
                                  for _ in range(4)))
        for q in range(4):
            accb[pl.ds(q * 16, 16)] = acc[q]
        pltpu.sync_copy(accb, ssum.at[sid])
        return car
    lax.fori_loop(0, 8, _one, 0)


def _sc_sheet(h3, sni):
    mesh = plsc.VectorSubcoreMesh(core_axis_name="c", subcore_axis_name="s")
    f = pl.kernel(
        _sheet_body,
        compiler_params=pltpu.CompilerParams(use_tc_tiling_on_sc=False),
        out_type=jax.ShapeDtypeStruct((256, D), jnp.float32),
        mesh=mesh,
        scratch_types=[
            pltpu.VMEM((4, 128), jnp.int32),
            pltpu.VMEM((512, D), jnp.float32),
            pltpu.VMEM((D,), jnp.float32),
            pltpu.SemaphoreType.DMA,
        ],
    )
    return f(h3, sni)


def _ln(x, g, b):
    mu = x.mean(-1, keepdims=True)
    var = x.var(-1, keepdims=True)
    return (x - mu) / jnp.sqrt(var + 1e-5) * g + b


def _heads_body(ss_ref, gs_ref, sf_ref,
                gW1, gb1, gg1, gbe1, gW2, gb2, gg2, gbe2,
                mWq, mbq, mWk, mbk, mWv, mbv, mWo, mbo,
                qW1, qb1, qg1, qbe1, qW2, qb2, qg2, qbe2, qW3, qb3,
                dW1, db1, dg1, dbe1, dW2, db2,
                qv_ref, dl_ref):
    se = ss_ref[...] / 512.0
    ge = gs_ref[...] / float(N)
    sf = sf_ref[...]
    geo = jax.nn.relu(_ln(_dot(sf, gW1[...]) + gb1[...], gg1[...], gbe1[...]))
    geo = jax.nn.relu(_ln(_dot(geo, gW2[...]) + gb2[...], gg2[...], gbe2[...]))
    qh = _dot(se, mWq[...]) + mbq[...]
    kh = _dot(geo, mWk[...]) + mbk[...]
    vh = _dot(geo, mWv[...]) + mbv[...]
    outs = []
    scale = 1.0 / np.sqrt(CH)
    for h in range(HEADS):
        q1 = qh[:, h * CH:(h + 1) * CH]
        k1 = kh[:, h * CH:(h + 1) * CH]
        v1 = vh[:, h * CH:(h + 1) * CH]
        att = _dot(q1, k1.T) * scale
        att = jax.nn.softmax(att, axis=-1)
        outs.append(_dot(att, v1))
    o = jnp.concatenate(outs, axis=-1)
    se2 = _dot(o, mWo[...]) + mbo[...]
    hq = jnp.concatenate([se2, jnp.broadcast_to(ge, (se2.shape[0], D))], axis=1)
    hq = jax.nn.relu(_ln(_dot(hq, qW1[...]) + qb1[...], qg1[...], qbe1[...]))
    hq = jax.nn.relu(_ln(_dot(hq, qW2[...]) + qb2[...], qg2[...], qbe2[...]))
    qv = _dot(hq, qW3[...]) + qb3[...]
    qv_ref[...] = qv[:, 0]
    hd = jax.nn.relu(_ln(_dot(ge, dW1[...]) + db1[...], dg1[...], dbe1[...]))
    dl = _dot(hd, dW2[...], hi=True) + db2[...]
    dl_ref[...] = dl[:, 0]


def _heads_call(sheet_sum, g_sum, sheet_features, params):
    gp, mp, qp, dp = params['geo'], params['mha'], params['qmlp'], params['done']
    args = [sheet_sum, g_sum, sheet_features,
            gp['W1'], gp['b1'], gp['g1'], gp['be1'],
            gp['W2'], gp['b2'], gp['g2'], gp['be2'],
            mp['Wq'], mp['bq'], mp['Wk'], mp['bk'],
            mp['Wv'], mp['bv'], mp['Wo'], mp['bo'],
            qp['W1'], qp['b1'], qp['g1'], qp['be1'],
            qp['W2'], qp['b2'], qp['g2'], qp['be2'], qp['W3'], qp['b3'],
            dp['W1'], dp['b1'], dp['g1'], dp['be1'], dp['W2'], dp['b2']]
    args = [a.reshape(1, -1) if a.ndim == 1 else a for a in args]
    qv, dl = pl.pallas_call(
        _heads_body,
        out_shape=(jax.ShapeDtypeStruct((256,), jnp.float32),
                   jax.ShapeDtypeStruct((1,), jnp.float32)),
    )(*args)
    return qv, dl


def kernel(x, edge_index, batch, sheet_node_idx, sheet_features, params):
    pad = jnp.zeros((E_PAD - E,), jnp.int32)
    src2 = jnp.concatenate([edge_index[0], pad]).reshape(E_PAD // 128, 128)
    dst2 = jnp.concatenate([edge_index[1], pad]).reshape(E_PAD // 128, 128)
    sni = sheet_node_idx.reshape(256, 4, 128)

    xl0, xl1, xr0, xr1 = _prep1(x, params['pre_W'], params['pre_b'],
                                params['gat'][0])
    h3 = gsum = None
    for l in range(3):
        p = params['gat'][l]
        attp = p['att'].reshape(2, 32)
        xl_all = jnp.concatenate([xl0, xl1], axis=0)
        xr_all = jnp.concatenate([xr0, xr1], axis=0)
        xlg, xrg = _sc_gather(src2, dst2, xl_all, xr_all)
        w, exw = _tc_edge(xlg, xrg, attp)
        out2 = _sc_scatter(dst2, w.reshape(2, E_PAD, WX), WX)
        den2 = _sc_scatter(dst2, exw.reshape(2, E_PAD, DX), DX)
        o0 = out2[0, :N, :]
        o1 = out2[1, :N, :]
        d0 = den2[0, :N, 0:2]
        d1 = den2[1, :N, 0:2]
        if l < 2:
            xl0, xl1, xr0, xr1 = _prep23(
                o0, o1, d0, d1, xl0, xl1, p['bias'],
                params['gat'][l + 1])
        else:
            h3, gsum = _final(o0, o1, d0, d1, xl0, xl1, p['bias'])

    ssum = _sc_sheet(h3, sni)
    q_values, state_done_logit = _heads_call(
        ssum, gsum, sheet_features, params)
    return q_values, state_done_logit
